# Initial kernel scaffold; baseline (speedup 1.0000x reference)
import jax
import jax.numpy as jnp
from jax.experimental import pallas as pl


def kernel(x1, x2):
    raise NotImplementedError("write your pallas kernel here")


# speedup vs baseline: 10.2095x; 10.2095x over previous
import jax
import jax.numpy as jnp
import numpy as np
from jax.experimental import pallas as pl

_P = 16
_NG = 24


def _pixel_mask_np() -> np.ndarray:
    total = _NG * _NG
    rand_pos = np.asarray(jax.random.permutation(jax.random.key(42), total))
    m = np.zeros(total, np.bool_)
    m[rand_pos[: total // 2]] = True
    m2 = m.reshape(_NG, _NG)
    return np.repeat(np.repeat(m2, _P, axis=0), _P, axis=1)


_MASK_NP = _pixel_mask_np().astype(np.float32)


def _select_body(m_ref, x1_ref, x2_ref, o_ref):
    o_ref[...] = jnp.where(m_ref[...] != 0.0, x2_ref[...], x1_ref[...])


def kernel(x1, x2):
    B, C, H, W = x1.shape
    N = B * C
    a = x1.reshape(N, H, W)
    b = x2.reshape(N, H, W)
    m = jnp.asarray(_MASK_NP).reshape(1, H, W)
    PB = 8
    out = pl.pallas_call(
        _select_body,
        grid=(N // PB,),
        in_specs=[
            pl.BlockSpec((1, H, W), lambda i: (0, 0, 0)),
            pl.BlockSpec((PB, H, W), lambda i: (i, 0, 0)),
            pl.BlockSpec((PB, H, W), lambda i: (i, 0, 0)),
        ],
        out_specs=pl.BlockSpec((PB, H, W), lambda i: (i, 0, 0)),
        out_shape=jax.ShapeDtypeStruct((N, H, W), x1.dtype),
    )(m, a, b)
    return out.reshape(B, C, H, W)
